# baseline (speedup 1.0000x reference)
import functools

import jax
import jax.numpy as jnp
from jax import lax
from jax.experimental import pallas as pl
from jax.experimental.pallas import tpu as pltpu
from jax.experimental.pallas import tpu_sc as plsc

_B = 16384
_F = 26
_D = 64
_NFLAT = _B * _F
_NW = 32
_ROWS_PER_W = _NFLAT // _NW
_CHUNK = 128
_NCHUNK = _ROWS_PER_W // _CHUNK


def _sc_gather(table, idx2d):
  mesh = plsc.VectorSubcoreMesh(core_axis_name="c", subcore_axis_name="s")

  @functools.partial(
      pl.kernel,
      out_type=jax.ShapeDtypeStruct((_NFLAT, 128), jnp.float32),
      mesh=mesh,
      scratch_types=[
          pltpu.VMEM((_NCHUNK, _CHUNK), jnp.int32),
          pltpu.VMEM((2, _CHUNK, _D), jnp.float32),
          pltpu.SemaphoreType.DMA,
          pltpu.SemaphoreType.DMA,
      ],
      compiler_params=pltpu.CompilerParams(use_tc_tiling_on_sc=False),
  )
  def k(table_hbm, idx_hbm, out_hbm, idx_v, rows_v, sem0, sem1):
    wid = lax.axis_index("s") * 2 + lax.axis_index("c")
    base_chunk = wid * _NCHUNK
    pltpu.sync_copy(idx_hbm.at[pl.ds(base_chunk, _NCHUNK)], idx_v)

    def store(slot, j):
      pltpu.sync_copy(
          rows_v.at[slot],
          out_hbm.at[pl.ds((base_chunk + j) * _CHUNK, _CHUNK), pl.ds(0, _D)])

    pltpu.async_copy(table_hbm.at[idx_v.at[0]], rows_v.at[0], sem0)

    def body2(i, _):
      j0 = i * 2
      @pl.when(j0 + 1 < _NCHUNK)
      def _():
        pltpu.async_copy(table_hbm.at[idx_v.at[j0 + 1]], rows_v.at[1], sem1)
      pltpu.make_async_copy(table_hbm.at[idx_v.at[j0]], rows_v.at[0],
                            sem0).wait()
      store(0, j0)
      @pl.when(j0 + 2 < _NCHUNK)
      def _():
        pltpu.async_copy(table_hbm.at[idx_v.at[j0 + 2]], rows_v.at[0], sem0)
      @pl.when(j0 + 1 < _NCHUNK)
      def _():
        pltpu.make_async_copy(table_hbm.at[idx_v.at[j0 + 1]], rows_v.at[1],
                              sem1).wait()
        store(1, j0 + 1)
      return _

    lax.fori_loop(0, _NCHUNK // 2, body2, None)

  return k(table, idx2d)


_BB = 128


def _mm_body(e_ref, w_ref, b_ref, o_ref):
  acc = jnp.dot(e_ref[:, :_D], w_ref[...], preferred_element_type=jnp.float32)
  y = jnp.maximum(acc + b_ref[0:1, :], 0.0)
  o_ref[...] = y.reshape(_BB, _F, _D)


def _tc_linear_relu(e, W, b):
  b2 = jnp.broadcast_to(b[None, :], (8, _D))
  grid = (_B // _BB,)
  return pl.pallas_call(
      _mm_body,
      grid=grid,
      in_specs=[
          pl.BlockSpec((_BB * _F, 128), lambda i: (i, 0)),
          pl.BlockSpec((_D, _D), lambda i: (0, 0)),
          pl.BlockSpec((8, _D), lambda i: (0, 0)),
      ],
      out_specs=pl.BlockSpec((_BB, _F, _D), lambda i: (i, 0, 0)),
      out_shape=jax.ShapeDtypeStruct((_B, _F, _D), jnp.float32),
  )(e, W, b2)


def kernel(x, table, W, b):
  idx2d = x.astype(jnp.int32).reshape(_NFLAT // _CHUNK, _CHUNK)
  rows = _sc_gather(table, idx2d)
  return _tc_linear_relu(rows, W, b)
